# Initial kernel scaffold; baseline (speedup 1.0000x reference)
import jax
import jax.numpy as jnp
from jax.experimental import pallas as pl


def kernel(x, edge_index, edge_attr, batch, params):
    raise NotImplementedError("write your pallas kernel here")


# speedup vs baseline: 1.0828x; 1.0828x over previous
import functools

import jax
import jax.numpy as jnp
from jax import lax
from jax.experimental import pallas as pl
from jax.experimental.pallas import tpu as pltpu
from jax.experimental.pallas import tpu_sc as plsc

N_NODES = 10000
N_EDGES = 20000
NF = 32
NE = 8
NGRAPHS = 256

NCORES = 2
NSUB = 16
NW = NCORES * NSUB
E_PAD = 20480
E_RPT = E_PAD // NW
N_PAD = 12288
N_RPT = N_PAD // NW
D = 128
NS0 = 5120
NS1 = N_NODES - NS0
NOUT_SPLIT = 5248
G_SOUT = 512


@functools.lru_cache(maxsize=None)
def _make_sc_gather(n_rows_pad):
    rpt = n_rows_pad // NW
    nchunk = rpt // 128
    mesh = plsc.VectorSubcoreMesh(core_axis_name="c", subcore_axis_name="s",
                                  num_cores=NCORES, num_subcores=NSUB)

    @functools.partial(
        pl.kernel,
        out_type=jax.ShapeDtypeStruct((n_rows_pad, D), jnp.float32),
        mesh=mesh,
        scratch_types=[
            pltpu.VMEM((nchunk, 128), jnp.int32),
            pltpu.VMEM((rpt, D), jnp.float32),
            pltpu.SemaphoreType.DMA,
        ],
    )
    def gather_k(table_hbm, idx3_hbm, out_hbm, idx_v, rows_v, sem):
        wid = lax.axis_index("c") * NSUB + lax.axis_index("s")
        pltpu.sync_copy(idx3_hbm.at[wid], idx_v)
        for j in range(nchunk):
            pltpu.async_copy(
                table_hbm.at[idx_v.at[j]],
                rows_v.at[pl.ds(j * 128, 128)], sem).wait()
        pltpu.sync_copy(rows_v, out_hbm.at[pl.ds(wid * rpt, rpt)])

    return gather_k


@functools.lru_cache(maxsize=None)
def _make_sc_scatter(n_rows_pad, n_out):
    rpt = n_rows_pad // NW
    nchunk = rpt // 128
    stripe = n_out // NSUB
    assert stripe % 8 == 0
    mesh = plsc.VectorSubcoreMesh(core_axis_name="c", subcore_axis_name="s",
                                  num_cores=NCORES, num_subcores=NSUB)

    @functools.partial(
        pl.kernel,
        out_type=jax.ShapeDtypeStruct((NCORES, n_out, D), jnp.float32),
        mesh=mesh,
        scratch_types=(
            [pltpu.VMEM((128,), jnp.int32) for _ in range(nchunk)]
            + [pltpu.VMEM((rpt, D), jnp.float32),
               pltpu.VMEM_SHARED((n_out, D), jnp.float32)]
        ),
    )
    def scatter_k(vals_hbm, idx2d_hbm, zeros_hbm, out_hbm, *scratch):
        idx_vs = scratch[:nchunk]
        vals_v = scratch[nchunk]
        acc_sh = scratch[nchunk + 1]
        c = lax.axis_index("c")
        s = lax.axis_index("s")
        wid = c * NSUB + s
        pltpu.sync_copy(zeros_hbm, acc_sh.at[pl.ds(s * stripe, stripe)])
        pltpu.sync_copy(vals_hbm.at[pl.ds(wid * rpt, rpt)], vals_v)
        for j in range(nchunk):
            pltpu.sync_copy(idx2d_hbm.at[wid * nchunk + j], idx_vs[j])
        plsc.subcore_barrier()
        for j in range(nchunk):
            pltpu.sync_copy(
                vals_v.at[pl.ds(j * 128, 128)],
                acc_sh.at[idx_vs[j]], add=True)
        plsc.subcore_barrier()
        pltpu.sync_copy(
            acc_sh.at[pl.ds(s * stripe, stripe)],
            out_hbm.at[c, pl.ds(s * stripe, stripe)])

    return scatter_k


def _make_tc_edge_msg(in_ch, out_ch, kk, e_blk):
    n_steps = E_PAD // e_blk

    def body(ea_ref, xs_ref, w1_ref, b1_ref, w2_ref, b2_ref, o_ref):
        h = jnp.maximum(
            jnp.dot(ea_ref[...], w1_ref[...],
                    preferred_element_type=jnp.float32) + b1_ref[...], 0.0)
        t = jnp.dot(h, w2_ref[...],
                    preferred_element_type=jnp.float32) + b2_ref[...]
        xs = xs_ref[...]
        acc = xs[:, 0:1] * t[:, 0:out_ch]
        for i in range(1, in_ch):
            acc = acc + xs[:, i:i + 1] * t[:, i * out_ch:(i + 1) * out_ch]
        row = (pl.program_id(0) * e_blk
               + lax.broadcasted_iota(jnp.int32, (e_blk, 1), 0))
        acc = jnp.where(row < N_EDGES, acc, 0.0)
        if out_ch < D:
            acc = jnp.concatenate(
                [acc, jnp.zeros((e_blk, D - out_ch), jnp.float32)], axis=1)
        o_ref[...] = acc

    def call(ea, xs, w1, b1, w2, b2):
        return pl.pallas_call(
            body,
            grid=(n_steps,),
            in_specs=[
                pl.BlockSpec((e_blk, NE), lambda i: (i, 0)),
                pl.BlockSpec((e_blk, D), lambda i: (i, 0)),
                pl.BlockSpec((NE, kk), lambda i: (0, 0)),
                pl.BlockSpec((1, kk), lambda i: (0, 0)),
                pl.BlockSpec((kk, in_ch * out_ch), lambda i: (0, 0)),
                pl.BlockSpec((1, in_ch * out_ch), lambda i: (0, 0)),
            ],
            out_specs=pl.BlockSpec((e_blk, D), lambda i: (i, 0)),
            out_shape=jax.ShapeDtypeStruct((E_PAD, D), jnp.float32),
        )(ea, xs, w1, b1, w2, b2)

    return call


def _tc_node_update(spA, spB, cntA, cntB, x, in_ch, root, bias, g, b,
                    attw, attb):
    out_ch = root.shape[1]

    def body(spA_ref, spB_ref, cA_ref, cB_ref, x_ref, root_ref, bias_ref,
             g_ref, b_ref, attw_ref, attb_ref, o_ref):
        s = jnp.concatenate(
            [spA_ref[0, 0:NS0, 0:out_ch] + spA_ref[1, 0:NS0, 0:out_ch],
             spB_ref[0, 0:NS1, 0:out_ch] + spB_ref[1, 0:NS1, 0:out_ch]],
            axis=0)
        cnt = jnp.concatenate(
            [cA_ref[0, 0:NS0, 0:1] + cA_ref[1, 0:NS0, 0:1],
             cB_ref[0, 0:NS1, 0:1] + cB_ref[1, 0:NS1, 0:1]], axis=0)
        agg = s / jnp.maximum(cnt, 1.0)
        xin = x_ref[:, 0:in_ch]
        hp = agg + jnp.dot(xin, root_ref[...],
                           preferred_element_type=jnp.float32) + bias_ref[...]
        m = jnp.mean(hp, axis=0, keepdims=True)
        hc = hp - m
        v = jnp.mean(hc * hc, axis=0, keepdims=True)
        hb = g_ref[...] * hc / jnp.sqrt(v + 1e-5) + b_ref[...]
        hr = jnp.maximum(hb, 0.0)
        a = jnp.dot(hr, attw_ref[...],
                    preferred_element_type=jnp.float32) + attb_ref[...]
        sig = 1.0 / (1.0 + jnp.exp(-a))
        out = hr * sig
        if out_ch < D:
            out = jnp.concatenate(
                [out, jnp.zeros((N_NODES, D - out_ch), jnp.float32)], axis=1)
        o_ref[...] = out

    return pl.pallas_call(
        body,
        out_shape=jax.ShapeDtypeStruct((N_NODES, D), jnp.float32),
    )(spA, spB, cntA, cntB, x, root, bias.reshape(1, out_ch),
      g.reshape(1, out_ch), b.reshape(1, out_ch), attw, attb.reshape(1, 1))


def _tc_pool_mlp(gp, gcnt, fc1w, fc1b, fc2w, fc2b):
    def body(gp_ref, gc_ref, w1_ref, b1_ref, w2_ref, b2_ref, o_ref):
        s = gp_ref[0, 0:NGRAPHS, 0:NF] + gp_ref[1, 0:NGRAPHS, 0:NF]
        cnt = gc_ref[0, 0:NGRAPHS, 0:1] + gc_ref[1, 0:NGRAPHS, 0:1]
        gmean = s / jnp.maximum(cnt, 1.0)
        h1 = jnp.maximum(
            jnp.dot(gmean, w1_ref[...],
                    preferred_element_type=jnp.float32) + b1_ref[...], 0.0)
        o_ref[...] = jnp.dot(h1, w2_ref[...],
                             preferred_element_type=jnp.float32) + b2_ref[...]

    return pl.pallas_call(
        body,
        out_shape=jax.ShapeDtypeStruct((NGRAPHS, 1), jnp.float32),
    )(gp, gcnt, fc1w, fc1b.reshape(1, -1), fc2w, fc2b.reshape(1, 1))


_edge1 = _make_tc_edge_msg(NF, 128, 128, 512)
_edge2 = _make_tc_edge_msg(128, 64, 128, 256)
_edge3 = _make_tc_edge_msg(64, 32, 64, 512)


def _nnconv_layer(edge_call, h_table, x_raw, in_ch, src3, dstA, dstB,
                  cntA, cntB, z_node, ea_p, p, pfx, bn_pfx, att_pfx):
    xs = _make_sc_gather(E_PAD)(h_table, src3)
    msg = edge_call(ea_p, xs, p[pfx + '_w1'], p[pfx + '_b1'].reshape(1, -1),
                    p[pfx + '_w2'], p[pfx + '_b2'].reshape(1, -1))
    scat = _make_sc_scatter(E_PAD, NOUT_SPLIT)
    spA = scat(msg, dstA, z_node)
    spB = scat(msg, dstB, z_node)
    cpfx = pfx.replace('nn', 'conv')
    return _tc_node_update(spA, spB, cntA, cntB, x_raw, in_ch,
                           p[cpfx + '_root'], p[cpfx + '_bias'],
                           p[bn_pfx + '_g'], p[bn_pfx + '_b'],
                           p[att_pfx + '_w'], p[att_pfx + '_b'])


def kernel(x, edge_index, edge_attr, batch, params):
    p = params
    src = edge_index[0]
    dst = edge_index[1]

    epad = E_PAD - N_EDGES
    src3 = jnp.pad(src, (0, epad)).reshape(NW, E_RPT // 128, 128)
    dst_p = jnp.pad(dst, (0, epad))
    dstA = jnp.where(dst_p < NS0, dst_p, NS0).reshape(E_PAD // 128, 128)
    dstB = jnp.where(dst_p >= NS0, dst_p - NS0, NS0).reshape(E_PAD // 128, 128)
    ea_p = jnp.pad(edge_attr, ((0, epad), (0, 0)))
    emask = (jnp.arange(E_PAD, dtype=jnp.int32) < N_EDGES).astype(jnp.float32)
    ones_e = jnp.broadcast_to(emask[:, None], (E_PAD, D))

    npad = N_PAD - N_NODES
    batch2d = jnp.pad(batch, (0, npad),
                      constant_values=NGRAPHS).reshape(N_PAD // 128, 128)
    nmask = (jnp.arange(N_PAD, dtype=jnp.int32) < N_NODES).astype(jnp.float32)
    ones_n = jnp.broadcast_to(nmask[:, None], (N_PAD, D))

    z_node = jnp.zeros((NOUT_SPLIT // NSUB, D), jnp.float32)
    z_g = jnp.zeros((G_SOUT // NSUB, D), jnp.float32)

    scat_e = _make_sc_scatter(E_PAD, NOUT_SPLIT)
    cntA = scat_e(ones_e, dstA, z_node)
    cntB = scat_e(ones_e, dstB, z_node)

    x_p128 = jnp.pad(x, ((0, 0), (0, D - NF)))
    h1 = _nnconv_layer(_edge1, x_p128, x, NF, src3, dstA, dstB, cntA, cntB,
                       z_node, ea_p, p, 'nn1', 'bn1', 'att1')
    h2 = _nnconv_layer(_edge2, h1, h1, 128, src3, dstA, dstB, cntA, cntB,
                       z_node, ea_p, p, 'nn2', 'bn2', 'att2')
    h3 = _nnconv_layer(_edge3, h2, h2, 64, src3, dstA, dstB, cntA, cntB,
                       z_node, ea_p, p, 'nn3', 'bn3', 'att3')

    scat_g = _make_sc_scatter(N_PAD, G_SOUT)
    gcnt = scat_g(ones_n, batch2d, z_g)
    h3_p = jnp.pad(h3, ((0, npad), (0, 0)))
    gp = scat_g(h3_p, batch2d, z_g)
    return _tc_pool_mlp(gp, gcnt, p['fc1_w'], p['fc1_b'],
                        p['fc2_w'], p['fc2_b'])


# speedup vs baseline: 2.2348x; 2.0640x over previous
import functools

import jax
import jax.numpy as jnp
from jax import lax
from jax.experimental import pallas as pl
from jax.experimental.pallas import tpu as pltpu
from jax.experimental.pallas import tpu_sc as plsc

N_NODES = 10000
N_EDGES = 20000
NF = 32
NE = 8
NGRAPHS = 256

NCORES = 2
NSUB = 16
NW = NCORES * NSUB
E_PAD = 20480
E_RPT = E_PAD // NW
N_PAD = 12288
N_RPT = N_PAD // NW
D = 128
NS0 = 5120
NS1 = N_NODES - NS0
NOUT_SPLIT = 5248
G_SOUT = 512


@functools.lru_cache(maxsize=None)
def _make_sc_gather(n_rows_pad):
    rpt = n_rows_pad // NW
    nchunk = rpt // 128
    mesh = plsc.VectorSubcoreMesh(core_axis_name="c", subcore_axis_name="s",
                                  num_cores=NCORES, num_subcores=NSUB)

    @functools.partial(
        pl.kernel,
        out_type=jax.ShapeDtypeStruct((n_rows_pad, D), jnp.float32),
        mesh=mesh,
        scratch_types=[
            pltpu.VMEM((nchunk, 128), jnp.int32),
            pltpu.VMEM((rpt, D), jnp.float32),
            pltpu.SemaphoreType.DMA,
        ],
    )
    def gather_k(table_hbm, idx3_hbm, out_hbm, idx_v, rows_v, sem):
        wid = lax.axis_index("c") * NSUB + lax.axis_index("s")
        pltpu.sync_copy(idx3_hbm.at[wid], idx_v)
        for j in range(nchunk):
            pltpu.async_copy(
                table_hbm.at[idx_v.at[j]],
                rows_v.at[pl.ds(j * 128, 128)], sem).wait()
        pltpu.sync_copy(rows_v, out_hbm.at[pl.ds(wid * rpt, rpt)])

    return gather_k


@functools.lru_cache(maxsize=None)
def _make_sc_scatter(n_rows_pad, n_out):
    rpt = n_rows_pad // NW
    nchunk = rpt // 128
    stripe = n_out // NSUB
    assert stripe % 8 == 0
    mesh = plsc.VectorSubcoreMesh(core_axis_name="c", subcore_axis_name="s",
                                  num_cores=NCORES, num_subcores=NSUB)

    @functools.partial(
        pl.kernel,
        out_type=jax.ShapeDtypeStruct((NCORES, n_out, D), jnp.float32),
        mesh=mesh,
        scratch_types=(
            [pltpu.VMEM((128,), jnp.int32) for _ in range(nchunk)]
            + [pltpu.VMEM((rpt, D), jnp.float32),
               pltpu.VMEM_SHARED((n_out, D), jnp.float32)]
        ),
    )
    def scatter_k(vals_hbm, idx2d_hbm, zeros_hbm, out_hbm, *scratch):
        idx_vs = scratch[:nchunk]
        vals_v = scratch[nchunk]
        acc_sh = scratch[nchunk + 1]
        c = lax.axis_index("c")
        s = lax.axis_index("s")
        wid = c * NSUB + s
        pltpu.sync_copy(zeros_hbm, acc_sh.at[pl.ds(s * stripe, stripe)])
        pltpu.sync_copy(vals_hbm.at[pl.ds(wid * rpt, rpt)], vals_v)
        for j in range(nchunk):
            pltpu.sync_copy(idx2d_hbm.at[wid * nchunk + j], idx_vs[j])
        plsc.subcore_barrier()
        for j in range(nchunk):
            pltpu.sync_copy(
                vals_v.at[pl.ds(j * 128, 128)],
                acc_sh.at[idx_vs[j]], add=True)
        plsc.subcore_barrier()
        pltpu.sync_copy(
            acc_sh.at[pl.ds(s * stripe, stripe)],
            out_hbm.at[c, pl.ds(s * stripe, stripe)])

    return scatter_k


def _make_tc_edge_msg(in_ch, out_ch, kk, e_blk):
    n_steps = E_PAD // e_blk
    io = in_ch * out_ch
    n_groups = io // D
    n_fold = D // out_ch

    def body(ea_ref, xs_ref, w1_ref, b1_ref, w2_ref, b2r_ref, rx_ref, o_ref):
        h = jnp.maximum(
            jnp.dot(ea_ref[...], w1_ref[...],
                    preferred_element_type=jnp.float32) + b1_ref[...], 0.0)
        t = jnp.dot(h, w2_ref[...], preferred_element_type=jnp.float32)
        xs = xs_ref[:, 0:in_ch]
        f = jnp.dot(xs, rx_ref[...], preferred_element_type=jnp.float32)
        p = f * t
        g = p[:, 0:D]
        for c in range(1, n_groups):
            g = g + p[:, c * D:(c + 1) * D]
        acc = g[:, 0:out_ch]
        for j in range(1, n_fold):
            acc = acc + g[:, j * out_ch:(j + 1) * out_ch]
        acc = acc + jnp.dot(xs, b2r_ref[...],
                            preferred_element_type=jnp.float32)
        row = (pl.program_id(0) * e_blk
               + lax.broadcasted_iota(jnp.int32, (e_blk, 1), 0))
        acc = jnp.where(row < N_EDGES, acc, 0.0)
        if out_ch < D:
            acc = jnp.concatenate(
                [acc, jnp.zeros((e_blk, D - out_ch), jnp.float32)], axis=1)
        o_ref[...] = acc

    def call(ea, xs, w1, b1, w2, b2):
        b2r = b2.reshape(in_ch, out_ch)
        rx = jnp.kron(jnp.eye(in_ch, dtype=jnp.float32),
                      jnp.ones((1, out_ch), jnp.float32))
        return pl.pallas_call(
            body,
            grid=(n_steps,),
            in_specs=[
                pl.BlockSpec((e_blk, NE), lambda i: (i, 0)),
                pl.BlockSpec((e_blk, D), lambda i: (i, 0)),
                pl.BlockSpec((NE, kk), lambda i: (0, 0)),
                pl.BlockSpec((1, kk), lambda i: (0, 0)),
                pl.BlockSpec((kk, io), lambda i: (0, 0)),
                pl.BlockSpec((in_ch, out_ch), lambda i: (0, 0)),
                pl.BlockSpec((in_ch, io), lambda i: (0, 0)),
            ],
            out_specs=pl.BlockSpec((e_blk, D), lambda i: (i, 0)),
            out_shape=jax.ShapeDtypeStruct((E_PAD, D), jnp.float32),
        )(ea, xs, w1, b1, w2, b2r, rx)

    return call


def _tc_node_update(spA, spB, cntA, cntB, x, in_ch, root, bias, g, b,
                    attw, attb):
    out_ch = root.shape[1]

    def body(spA_ref, spB_ref, cA_ref, cB_ref, x_ref, root_ref, bias_ref,
             g_ref, b_ref, attw_ref, attb_ref, o_ref):
        s = jnp.concatenate(
            [spA_ref[0, 0:NS0, 0:out_ch] + spA_ref[1, 0:NS0, 0:out_ch],
             spB_ref[0, 0:NS1, 0:out_ch] + spB_ref[1, 0:NS1, 0:out_ch]],
            axis=0)
        cnt = jnp.concatenate(
            [cA_ref[0, 0:NS0, 0:1] + cA_ref[1, 0:NS0, 0:1],
             cB_ref[0, 0:NS1, 0:1] + cB_ref[1, 0:NS1, 0:1]], axis=0)
        agg = s / jnp.maximum(cnt, 1.0)
        xin = x_ref[:, 0:in_ch]
        hp = agg + jnp.dot(xin, root_ref[...],
                           preferred_element_type=jnp.float32) + bias_ref[...]
        m = jnp.mean(hp, axis=0, keepdims=True)
        hc = hp - m
        v = jnp.mean(hc * hc, axis=0, keepdims=True)
        hb = g_ref[...] * hc / jnp.sqrt(v + 1e-5) + b_ref[...]
        hr = jnp.maximum(hb, 0.0)
        a = jnp.dot(hr, attw_ref[...],
                    preferred_element_type=jnp.float32) + attb_ref[...]
        sig = 1.0 / (1.0 + jnp.exp(-a))
        out = hr * sig
        if out_ch < D:
            out = jnp.concatenate(
                [out, jnp.zeros((N_NODES, D - out_ch), jnp.float32)], axis=1)
        o_ref[...] = out

    return pl.pallas_call(
        body,
        out_shape=jax.ShapeDtypeStruct((N_NODES, D), jnp.float32),
    )(spA, spB, cntA, cntB, x, root, bias.reshape(1, out_ch),
      g.reshape(1, out_ch), b.reshape(1, out_ch), attw, attb.reshape(1, 1))


def _tc_pool_mlp(gp, gcnt, fc1w, fc1b, fc2w, fc2b):
    def body(gp_ref, gc_ref, w1_ref, b1_ref, w2_ref, b2_ref, o_ref):
        s = gp_ref[0, 0:NGRAPHS, 0:NF] + gp_ref[1, 0:NGRAPHS, 0:NF]
        cnt = gc_ref[0, 0:NGRAPHS, 0:1] + gc_ref[1, 0:NGRAPHS, 0:1]
        gmean = s / jnp.maximum(cnt, 1.0)
        h1 = jnp.maximum(
            jnp.dot(gmean, w1_ref[...],
                    preferred_element_type=jnp.float32) + b1_ref[...], 0.0)
        o_ref[...] = jnp.dot(h1, w2_ref[...],
                             preferred_element_type=jnp.float32) + b2_ref[...]

    return pl.pallas_call(
        body,
        out_shape=jax.ShapeDtypeStruct((NGRAPHS, 1), jnp.float32),
    )(gp, gcnt, fc1w, fc1b.reshape(1, -1), fc2w, fc2b.reshape(1, 1))


_edge1 = _make_tc_edge_msg(NF, 128, 128, 512)
_edge2 = _make_tc_edge_msg(128, 64, 128, 256)
_edge3 = _make_tc_edge_msg(64, 32, 64, 512)


def _nnconv_layer(edge_call, h_table, x_raw, in_ch, src3, dstA, dstB,
                  cntA, cntB, z_node, ea_p, p, pfx, bn_pfx, att_pfx):
    xs = _make_sc_gather(E_PAD)(h_table, src3)
    msg = edge_call(ea_p, xs, p[pfx + '_w1'], p[pfx + '_b1'].reshape(1, -1),
                    p[pfx + '_w2'], p[pfx + '_b2'].reshape(1, -1))
    scat = _make_sc_scatter(E_PAD, NOUT_SPLIT)
    spA = scat(msg, dstA, z_node)
    spB = scat(msg, dstB, z_node)
    cpfx = pfx.replace('nn', 'conv')
    return _tc_node_update(spA, spB, cntA, cntB, x_raw, in_ch,
                           p[cpfx + '_root'], p[cpfx + '_bias'],
                           p[bn_pfx + '_g'], p[bn_pfx + '_b'],
                           p[att_pfx + '_w'], p[att_pfx + '_b'])


def kernel(x, edge_index, edge_attr, batch, params):
    p = params
    src = edge_index[0]
    dst = edge_index[1]

    epad = E_PAD - N_EDGES
    src3 = jnp.pad(src, (0, epad)).reshape(NW, E_RPT // 128, 128)
    dst_p = jnp.pad(dst, (0, epad))
    dstA = jnp.where(dst_p < NS0, dst_p, NS0).reshape(E_PAD // 128, 128)
    dstB = jnp.where(dst_p >= NS0, dst_p - NS0, NS0).reshape(E_PAD // 128, 128)
    ea_p = jnp.pad(edge_attr, ((0, epad), (0, 0)))
    emask = (jnp.arange(E_PAD, dtype=jnp.int32) < N_EDGES).astype(jnp.float32)
    ones_e = jnp.broadcast_to(emask[:, None], (E_PAD, D))

    npad = N_PAD - N_NODES
    batch2d = jnp.pad(batch, (0, npad),
                      constant_values=NGRAPHS).reshape(N_PAD // 128, 128)
    nmask = (jnp.arange(N_PAD, dtype=jnp.int32) < N_NODES).astype(jnp.float32)
    ones_n = jnp.broadcast_to(nmask[:, None], (N_PAD, D))

    z_node = jnp.zeros((NOUT_SPLIT // NSUB, D), jnp.float32)
    z_g = jnp.zeros((G_SOUT // NSUB, D), jnp.float32)

    scat_e = _make_sc_scatter(E_PAD, NOUT_SPLIT)
    cntA = scat_e(ones_e, dstA, z_node)
    cntB = scat_e(ones_e, dstB, z_node)

    x_p128 = jnp.pad(x, ((0, 0), (0, D - NF)))
    h1 = _nnconv_layer(_edge1, x_p128, x, NF, src3, dstA, dstB, cntA, cntB,
                       z_node, ea_p, p, 'nn1', 'bn1', 'att1')
    h2 = _nnconv_layer(_edge2, h1, h1, 128, src3, dstA, dstB, cntA, cntB,
                       z_node, ea_p, p, 'nn2', 'bn2', 'att2')
    h3 = _nnconv_layer(_edge3, h2, h2, 64, src3, dstA, dstB, cntA, cntB,
                       z_node, ea_p, p, 'nn3', 'bn3', 'att3')

    scat_g = _make_sc_scatter(N_PAD, G_SOUT)
    gcnt = scat_g(ones_n, batch2d, z_g)
    h3_p = jnp.pad(h3, ((0, npad), (0, 0)))
    gp = scat_g(h3_p, batch2d, z_g)
    return _tc_pool_mlp(gp, gcnt, p['fc1_w'], p['fc1_b'],
                        p['fc2_w'], p['fc2_b'])
